# baseline (speedup 1.0000x reference)
import jax
import jax.numpy as jnp
from jax import lax
from jax.experimental import pallas as pl
from jax.experimental.pallas import tpu as pltpu
from jax.experimental.pallas import tpu_sc as plsc

_NT = 32768
_H = 768
_NE = 64
_BT = 4096

_CH = 2
_CT = _NT // _CH
_NW = 32
_TPW = _CT // _NW
_IL = 4
_G = _TPW // (16 * _IL)


def _matmul_body(x_ref, w_ref, lg_ref):
    lg_ref[...] = jax.lax.dot_general(
        w_ref[...], x_ref[...],
        dimension_numbers=(((1,), (1,)), ((), ())),
        preferred_element_type=jnp.float32)


def _tc_logits_t(x, W):
    return pl.pallas_call(
        _matmul_body,
        grid=(_CT // _BT,),
        in_specs=[
            pl.BlockSpec((_BT, _H), lambda i: (i, 0)),
            pl.BlockSpec((_NE, _H), lambda i: (0, 0)),
        ],
        out_specs=pl.BlockSpec((_NE, _BT), lambda i: (0, i)),
        out_shape=jax.ShapeDtypeStruct((_NE, _CT), jnp.float32),
        compiler_params=pltpu.CompilerParams(
            dimension_semantics=("arbitrary",)),
    )(x, W)


def _route_body(lg_hbm, w_hbm, e_hbm, buf, w1b, w2b, e1b, e2b):
    wid = lax.axis_index("s") * 2 + lax.axis_index("c")
    base = wid * _TPW
    pltpu.sync_copy(lg_hbm.at[:, pl.ds(base, _TPW)], buf)

    def block(g, carry):
        off = g * (16 * _IL)
        m1 = [buf[0, pl.ds(off + 16 * j, 16)] for j in range(_IL)]
        i1 = [jnp.zeros((16,), jnp.int32) for _ in range(_IL)]
        m2 = [jnp.full((16,), -jnp.inf, jnp.float32) for _ in range(_IL)]
        i2 = [jnp.zeros((16,), jnp.int32) for _ in range(_IL)]
        for e in range(1, _NE):
            ev = jnp.full((16,), e, jnp.int32)
            for j in range(_IL):
                v = buf[e, pl.ds(off + 16 * j, 16)]
                c1 = v > m1[j]
                c2 = v > m2[j]
                m2[j] = jnp.where(c1, m1[j], jnp.where(c2, v, m2[j]))
                i2[j] = jnp.where(c1, i1[j], jnp.where(c2, ev, i2[j]))
                m1[j] = jnp.where(c1, v, m1[j])
                i1[j] = jnp.where(c1, ev, i1[j])
        for j in range(_IL):
            t = jnp.exp(m2[j] - m1[j])
            d = 1.0 + t
            sl = pl.ds(off + 16 * j, 16)
            w1b[sl] = 1.0 / d
            w2b[sl] = t / d
            e1b[sl] = i1[j]
            e2b[sl] = i2[j]
        return carry

    lax.fori_loop(0, _G, block, 0)
    pltpu.sync_copy(w1b, w_hbm.at[0, pl.ds(base, _TPW)])
    pltpu.sync_copy(w2b, w_hbm.at[1, pl.ds(base, _TPW)])
    pltpu.sync_copy(e1b, e_hbm.at[0, pl.ds(base, _TPW)])
    pltpu.sync_copy(e2b, e_hbm.at[1, pl.ds(base, _TPW)])


_route = pl.kernel(
    _route_body,
    out_type=[
        jax.ShapeDtypeStruct((2, _CT), jnp.float32),
        jax.ShapeDtypeStruct((2, _CT), jnp.int32),
    ],
    mesh=plsc.VectorSubcoreMesh(core_axis_name="c", subcore_axis_name="s"),
    compiler_params=pltpu.CompilerParams(needs_layout_passes=False),
    scratch_types=[
        pltpu.VMEM((_NE, _TPW), jnp.float32),
        pltpu.VMEM((_TPW,), jnp.float32),
        pltpu.VMEM((_TPW,), jnp.float32),
        pltpu.VMEM((_TPW,), jnp.int32),
        pltpu.VMEM((_TPW,), jnp.int32),
    ],
)


def kernel(x, W):
    parts = []
    for c in range(_CH):
        lg = _tc_logits_t(lax.slice_in_dim(x, c * _CT, (c + 1) * _CT), W)
        parts.append(_route(lg))
    rw_t = jnp.concatenate([p[0] for p in parts], axis=1)
    se_t = jnp.concatenate([p[1] for p in parts], axis=1)
    return (rw_t.T, se_t.T)


# speedup vs baseline: 1.9055x; 1.9055x over previous
import jax
import jax.numpy as jnp
from jax import lax
from jax.experimental import pallas as pl
from jax.experimental.pallas import tpu as pltpu
from jax.experimental.pallas import tpu_sc as plsc

_NT = 32768
_H = 768
_NE = 64
_BT = 4096

_CH = 2
_CT = _NT // _CH
_NW = 32
_TPW = _CT // _NW
_IL = 4
_G = _TPW // (16 * _IL)


def _matmul_body(x_ref, w_ref, lg_ref):
    lg_ref[...] = jax.lax.dot_general(
        w_ref[...], x_ref[...],
        dimension_numbers=(((1,), (1,)), ((), ())),
        preferred_element_type=jnp.float32)


def _tc_logits_t(x, W, c):
    off = c * (_CT // _BT)
    return pl.pallas_call(
        _matmul_body,
        grid=(_CT // _BT,),
        in_specs=[
            pl.BlockSpec((_BT, _H), lambda i: (i + off, 0)),
            pl.BlockSpec((_NE, _H), lambda i: (0, 0)),
        ],
        out_specs=pl.BlockSpec((_NE, _BT), lambda i: (0, i)),
        out_shape=jax.ShapeDtypeStruct((_NE, _CT), jnp.float32),
        compiler_params=pltpu.CompilerParams(
            dimension_semantics=("arbitrary",)),
    )(x, W)


def _route_body(lg_hbm, w_hbm, e_hbm, buf, w1b, w2b, e1b, e2b):
    wid = lax.axis_index("s") * 2 + lax.axis_index("c")
    base = wid * _TPW
    pltpu.sync_copy(lg_hbm.at[:, pl.ds(base, _TPW)], buf)

    def block(g, carry):
        off = g * (16 * _IL)
        m1 = [buf[0, pl.ds(off + 16 * j, 16)] for j in range(_IL)]
        i1 = [jnp.zeros((16,), jnp.int32) for _ in range(_IL)]
        m2 = [jnp.full((16,), -jnp.inf, jnp.float32) for _ in range(_IL)]
        i2 = [jnp.zeros((16,), jnp.int32) for _ in range(_IL)]
        for e in range(1, _NE):
            ev = jnp.full((16,), e, jnp.int32)
            for j in range(_IL):
                v = buf[e, pl.ds(off + 16 * j, 16)]
                c1 = v > m1[j]
                c2 = v > m2[j]
                m2[j] = jnp.where(c1, m1[j], jnp.where(c2, v, m2[j]))
                i2[j] = jnp.where(c1, i1[j], jnp.where(c2, ev, i2[j]))
                m1[j] = jnp.where(c1, v, m1[j])
                i1[j] = jnp.where(c1, ev, i1[j])
        for j in range(_IL):
            t = jnp.exp(m2[j] - m1[j])
            d = 1.0 + t
            sl = pl.ds(off + 16 * j, 16)
            w1b[sl] = 1.0 / d
            w2b[sl] = t / d
            e1b[sl] = i1[j]
            e2b[sl] = i2[j]
        return carry

    lax.fori_loop(0, _G, block, 0)
    pltpu.sync_copy(w1b, w_hbm.at[0, pl.ds(base, _TPW)])
    pltpu.sync_copy(w2b, w_hbm.at[1, pl.ds(base, _TPW)])
    pltpu.sync_copy(e1b, e_hbm.at[0, pl.ds(base, _TPW)])
    pltpu.sync_copy(e2b, e_hbm.at[1, pl.ds(base, _TPW)])


_route = pl.kernel(
    _route_body,
    out_type=[
        jax.ShapeDtypeStruct((2, _CT), jnp.float32),
        jax.ShapeDtypeStruct((2, _CT), jnp.int32),
    ],
    mesh=plsc.VectorSubcoreMesh(core_axis_name="c", subcore_axis_name="s"),
    compiler_params=pltpu.CompilerParams(needs_layout_passes=False),
    scratch_types=[
        pltpu.VMEM((_NE, _TPW), jnp.float32),
        pltpu.VMEM((_TPW,), jnp.float32),
        pltpu.VMEM((_TPW,), jnp.float32),
        pltpu.VMEM((_TPW,), jnp.int32),
        pltpu.VMEM((_TPW,), jnp.int32),
    ],
)


def kernel(x, W):
    parts = []
    for c in range(_CH):
        lg = _tc_logits_t(x, W, c)
        parts.append(_route(lg))
    rw_t = jnp.concatenate([p[0] for p in parts], axis=1)
    se_t = jnp.concatenate([p[1] for p in parts], axis=1)
    return (rw_t.T, se_t.T)


# speedup vs baseline: 3.7852x; 1.9864x over previous
import jax
import jax.numpy as jnp
from jax.experimental import pallas as pl
from jax.experimental.pallas import tpu as pltpu

_NT = 32768
_H = 768
_NE = 64
_BT = 4096


def _body(x_ref, w_ref, rw_ref, se_ref):
    logits = jax.lax.dot_general(
        w_ref[...], x_ref[...],
        dimension_numbers=(((1,), (1,)), ((), ())),
        preferred_element_type=jnp.float32)
    e_ids = jax.lax.broadcasted_iota(jnp.int32, logits.shape, 0)
    m1 = jnp.max(logits, axis=0, keepdims=True)
    i1 = jnp.min(jnp.where(logits == m1, e_ids, _NE), axis=0, keepdims=True)
    masked = jnp.where(e_ids == i1, -jnp.inf, logits)
    m2 = jnp.max(masked, axis=0, keepdims=True)
    i2 = jnp.min(jnp.where(masked == m2, e_ids, _NE), axis=0, keepdims=True)
    t = jnp.exp(m2 - m1)
    d = 1.0 + t
    rw_ref[...] = jnp.concatenate([1.0 / d, t / d], axis=0)
    se_ref[...] = jnp.concatenate([i1, i2], axis=0)


def kernel(x, W):
    rw_t, se_t = pl.pallas_call(
        _body,
        grid=(_NT // _BT,),
        in_specs=[
            pl.BlockSpec((_BT, _H), lambda i: (i, 0)),
            pl.BlockSpec((_NE, _H), lambda i: (0, 0)),
        ],
        out_specs=[
            pl.BlockSpec((2, _BT), lambda i: (0, i)),
            pl.BlockSpec((2, _BT), lambda i: (0, i)),
        ],
        out_shape=[
            jax.ShapeDtypeStruct((2, _NT), jnp.float32),
            jax.ShapeDtypeStruct((2, _NT), jnp.int32),
        ],
        compiler_params=pltpu.CompilerParams(
            dimension_semantics=("arbitrary",)),
    )(x, W)
    return (rw_t.T, se_t.T)
